# baseline (speedup 1.0000x reference)
import functools

import jax
import jax.numpy as jnp
from jax import lax
from jax.experimental import pallas as pl
from jax.experimental.pallas import tpu as pltpu
from jax.experimental.pallas import tpu_sc as plsc

N = 10000
N_PAD = 10112
TRASH = 10008
E = 320000
N_OUT = 64
K = 5
ALPHA = 0.1
NW = 32
B = 128
NB = 80
NBUF = 4
NT = NB // NBUF
E_PAD = NW * NB * B
ROWS_PER_TILE = N_PAD // 16

_mesh = plsc.VectorSubcoreMesh(core_axis_name="c", subcore_axis_name="s")


def _mlp_body(x_ref, w1_ref, b1_ref, w2_ref, b2_ref, o_ref):
    h = jnp.maximum(
        jnp.dot(x_ref[...], w1_ref[...], preferred_element_type=jnp.float32)
        + b1_ref[...],
        0.0,
    )
    o_ref[...] = (
        jnp.dot(h, w2_ref[...], preferred_element_type=jnp.float32) + b2_ref[...]
    )


def _mlp(x, w1t, b1, w2t, b2):
    blk = 1000
    grid = N // blk
    return pl.pallas_call(
        _mlp_body,
        grid=(grid,),
        in_specs=[
            pl.BlockSpec((blk, 128), lambda i: (i, 0)),
            pl.BlockSpec((128, 128), lambda i: (0, 0)),
            pl.BlockSpec((1, 128), lambda i: (0, 0)),
            pl.BlockSpec((128, 64), lambda i: (0, 0)),
            pl.BlockSpec((1, 64), lambda i: (0, 0)),
        ],
        out_specs=pl.BlockSpec((blk, 64), lambda i: (i, 0)),
        out_shape=jax.ShapeDtypeStruct((N, 64), jnp.float32),
    )(x, w1t, b1, w2t, b2)


def _deg_body(dst_hbm, zeros_hbm, ones_hbm, out_hbm, dst_v, ones_v, deg_sp, sem):
    c = lax.axis_index("c")
    s = lax.axis_index("s")
    wid = c * 16 + s
    r0 = s * ROWS_PER_TILE
    pltpu.sync_copy(zeros_hbm.at[pl.ds(r0, ROWS_PER_TILE)],
                    deg_sp.at[pl.ds(r0, ROWS_PER_TILE)])
    pltpu.sync_copy(dst_hbm.at[wid], dst_v)
    pltpu.sync_copy(ones_hbm, ones_v)
    plsc.subcore_barrier()

    def blk(j, carry):
        pltpu.async_copy(ones_v, deg_sp.at[dst_v.at[j]], sem, add=True)
        return carry

    lax.fori_loop(0, NB, blk, 0, unroll=False)

    def drain(j, carry):
        pltpu.make_async_copy(ones_v, deg_sp.at[dst_v.at[j]], sem).wait()
        return carry

    lax.fori_loop(0, NB, drain, 0, unroll=False)
    plsc.subcore_barrier()
    pltpu.sync_copy(deg_sp.at[pl.ds(r0, ROWS_PER_TILE)],
                    out_hbm.at[c, pl.ds(r0, ROWS_PER_TILE)])


@functools.partial(
    pl.kernel,
    out_type=jax.ShapeDtypeStruct((2, N_PAD, 16), jnp.float32),
    mesh=_mesh,
    compiler_params=pltpu.CompilerParams(use_tc_tiling_on_sc=False),
    scratch_types=[
        pltpu.VMEM((NB, B), jnp.int32),
        pltpu.VMEM((B, 16), jnp.float32),
        pltpu.VMEM_SHARED((N_PAD, 16), jnp.float32),
        pltpu.SemaphoreType.DMA,
    ],
)
def _deg_sc(dst_hbm, zeros_hbm, ones_hbm, out_hbm, dst_v, ones_v, deg_sp, sem):
    _deg_body(dst_hbm, zeros_hbm, ones_hbm, out_hbm, dst_v, ones_v, deg_sp, sem)


def _dinv_body(d0_ref, d1_ref, h_ref, dinv_ref, u_ref):
    deg = d0_ref[...] + d1_ref[...] + 1.0
    dinv = lax.rsqrt(deg)
    dinv_ref[...] = dinv
    u_ref[...] = dinv * h_ref[...]


def _dinv_u0(deg0, deg1, h):
    blk = 1000
    grid = N // blk
    return pl.pallas_call(
        _dinv_body,
        grid=(grid,),
        in_specs=[
            pl.BlockSpec((blk, 1), lambda i: (i, 0)),
            pl.BlockSpec((blk, 1), lambda i: (i, 0)),
            pl.BlockSpec((blk, 64), lambda i: (i, 0)),
        ],
        out_specs=[
            pl.BlockSpec((blk, 1), lambda i: (i, 0)),
            pl.BlockSpec((blk, 64), lambda i: (i, 0)),
        ],
        out_shape=[
            jax.ShapeDtypeStruct((N, 1), jnp.float32),
            jax.ShapeDtypeStruct((N, 64), jnp.float32),
        ],
    )(deg0, deg1, h)


def _prop_body(u_hbm, src_hbm, dst_hbm, zeros_hbm, out_hbm,
               src_v, dst_v, b0, b1, b2, b3, s_sp, m0, m1, m2, m3):
    bufs = (b0, b1, b2, b3)
    sems = (m0, m1, m2, m3)
    c = lax.axis_index("c")
    s = lax.axis_index("s")
    wid = c * 16 + s
    r0 = s * ROWS_PER_TILE
    pltpu.sync_copy(zeros_hbm.at[pl.ds(r0, ROWS_PER_TILE)],
                    s_sp.at[pl.ds(r0, ROWS_PER_TILE)])
    pltpu.sync_copy(src_hbm.at[wid], src_v)
    pltpu.sync_copy(dst_hbm.at[wid], dst_v)
    plsc.subcore_barrier()

    def blk(j, carry):
        pltpu.sync_copy(u_hbm.at[src_v.at[j]], b0)
        pltpu.sync_copy(b0, s_sp.at[dst_v.at[j]], add=True)
        return carry

    lax.fori_loop(0, NB, blk, 0, unroll=False)

    plsc.subcore_barrier()
    pltpu.sync_copy(s_sp.at[pl.ds(r0, ROWS_PER_TILE)],
                    out_hbm.at[c, pl.ds(r0, ROWS_PER_TILE)])


@functools.partial(
    pl.kernel,
    out_type=jax.ShapeDtypeStruct((2, N_PAD, N_OUT), jnp.float32),
    mesh=_mesh,
    compiler_params=pltpu.CompilerParams(use_tc_tiling_on_sc=False),
    scratch_types=[
        pltpu.VMEM((NB, B), jnp.int32),
        pltpu.VMEM((NB, B), jnp.int32),
        pltpu.VMEM((B, N_OUT), jnp.float32),
        pltpu.VMEM((B, N_OUT), jnp.float32),
        pltpu.VMEM((B, N_OUT), jnp.float32),
        pltpu.VMEM((B, N_OUT), jnp.float32),
        pltpu.VMEM_SHARED((N_PAD, N_OUT), jnp.float32),
        pltpu.SemaphoreType.DMA,
        pltpu.SemaphoreType.DMA,
        pltpu.SemaphoreType.DMA,
        pltpu.SemaphoreType.DMA,
    ],
)
def _prop_sc(u_hbm, src_hbm, dst_hbm, zeros_hbm, out_hbm,
             src_v, dst_v, b0, b1, b2, b3, s_sp, m0, m1, m2, m3):
    _prop_body(u_hbm, src_hbm, dst_hbm, zeros_hbm, out_hbm,
               src_v, dst_v, b0, b1, b2, b3, s_sp, m0, m1, m2, m3)


def _comb_body(s0_ref, s1_ref, u_ref, h_ref, dinv_ref, unew_ref, onew_ref):
    s = s0_ref[...] + s1_ref[...] + u_ref[...]
    onew = (1.0 - ALPHA) * dinv_ref[...] * s + ALPHA * h_ref[...]
    onew_ref[...] = onew
    unew_ref[...] = dinv_ref[...] * onew


def _combine(s0, s1, u, h, dinv):
    blk = 1000
    grid = N // blk
    return pl.pallas_call(
        _comb_body,
        grid=(grid,),
        in_specs=[
            pl.BlockSpec((blk, 64), lambda i: (i, 0)),
            pl.BlockSpec((blk, 64), lambda i: (i, 0)),
            pl.BlockSpec((blk, 64), lambda i: (i, 0)),
            pl.BlockSpec((blk, 64), lambda i: (i, 0)),
            pl.BlockSpec((blk, 1), lambda i: (i, 0)),
        ],
        out_specs=[
            pl.BlockSpec((blk, 64), lambda i: (i, 0)),
            pl.BlockSpec((blk, 64), lambda i: (i, 0)),
        ],
        out_shape=[
            jax.ShapeDtypeStruct((N, 64), jnp.float32),
            jax.ShapeDtypeStruct((N, 64), jnp.float32),
        ],
    )(s0, s1, u, h, dinv)


def kernel(x, edge_index, W1, b1, W2, b2):
    src = edge_index[0]
    dst = edge_index[1]
    pad = E_PAD - E
    src_p = jnp.concatenate([src, jnp.zeros((pad,), jnp.int32)])
    dst_p = jnp.concatenate([dst, jnp.full((pad,), TRASH, jnp.int32)])
    src_b = src_p.reshape(NW, NB, B)
    dst_b = dst_p.reshape(NW, NB, B)

    zeros64 = jnp.zeros((N_PAD, N_OUT), jnp.float32)
    zeros16 = jnp.zeros((N_PAD, 16), jnp.float32)
    ones16 = jnp.ones((B, 16), jnp.float32)

    h = _mlp(x, W1.T, b1.reshape(1, -1), W2.T, b2.reshape(1, -1))

    degp = _deg_sc(dst_b, zeros16, ones16)
    deg0 = degp[0, :N, :1]
    deg1 = degp[1, :N, :1]

    dinv, u = _dinv_u0(deg0, deg1, h)

    out = None
    for _ in range(K):
        sp = _prop_sc(u, src_b, dst_b, zeros64)
        u, out = _combine(sp[0, :N], sp[1, :N], u, h, dinv)
    return out


# speedup vs baseline: 1.1447x; 1.1447x over previous
import functools

import jax
import jax.numpy as jnp
from jax import lax
from jax.experimental import pallas as pl
from jax.experimental.pallas import tpu as pltpu
from jax.experimental.pallas import tpu_sc as plsc

N = 10000
N_PAD = 10112
TRASH = 10008
E = 320000
N_OUT = 64
K = 5
ALPHA = 0.1
NW = 32
B = 128
NB = 80
NBUF = 4
NT = NB // NBUF
E_PAD = NW * NB * B
ROWS_PER_TILE = N_PAD // 16

_mesh = plsc.VectorSubcoreMesh(core_axis_name="c", subcore_axis_name="s")


def _mlp_body(x_ref, w1_ref, b1_ref, w2_ref, b2_ref, o_ref):
    h = jnp.maximum(
        jnp.dot(x_ref[...], w1_ref[...], preferred_element_type=jnp.float32)
        + b1_ref[...],
        0.0,
    )
    o_ref[...] = (
        jnp.dot(h, w2_ref[...], preferred_element_type=jnp.float32) + b2_ref[...]
    )


def _mlp(x, w1t, b1, w2t, b2):
    blk = 1000
    grid = N // blk
    return pl.pallas_call(
        _mlp_body,
        grid=(grid,),
        in_specs=[
            pl.BlockSpec((blk, 128), lambda i: (i, 0)),
            pl.BlockSpec((128, 128), lambda i: (0, 0)),
            pl.BlockSpec((1, 128), lambda i: (0, 0)),
            pl.BlockSpec((128, 64), lambda i: (0, 0)),
            pl.BlockSpec((1, 64), lambda i: (0, 0)),
        ],
        out_specs=pl.BlockSpec((blk, 64), lambda i: (i, 0)),
        out_shape=jax.ShapeDtypeStruct((N, 64), jnp.float32),
    )(x, w1t, b1, w2t, b2)


def _deg_body(dst_hbm, zeros_hbm, ones_hbm, out_hbm, dst_v, ones_v, deg_sp, sem):
    c = lax.axis_index("c")
    s = lax.axis_index("s")
    wid = c * 16 + s
    r0 = s * ROWS_PER_TILE
    pltpu.sync_copy(zeros_hbm.at[pl.ds(r0, ROWS_PER_TILE)],
                    deg_sp.at[pl.ds(r0, ROWS_PER_TILE)])
    pltpu.sync_copy(dst_hbm.at[wid], dst_v)
    pltpu.sync_copy(ones_hbm, ones_v)
    plsc.subcore_barrier()

    def blk(j, carry):
        pltpu.sync_copy(ones_v, deg_sp.at[dst_v.at[j]], add=True)
        return carry

    lax.fori_loop(0, NB, blk, 0, unroll=False)
    plsc.subcore_barrier()
    pltpu.sync_copy(deg_sp.at[pl.ds(r0, ROWS_PER_TILE)],
                    out_hbm.at[c, pl.ds(r0, ROWS_PER_TILE)])


@functools.partial(
    pl.kernel,
    out_type=jax.ShapeDtypeStruct((2, N_PAD, 16), jnp.float32),
    mesh=_mesh,
    compiler_params=pltpu.CompilerParams(use_tc_tiling_on_sc=False),
    scratch_types=[
        pltpu.VMEM((NB, B), jnp.int32),
        pltpu.VMEM((B, 16), jnp.float32),
        pltpu.VMEM_SHARED((N_PAD, 16), jnp.float32),
        pltpu.SemaphoreType.DMA,
    ],
)
def _deg_sc(dst_hbm, zeros_hbm, ones_hbm, out_hbm, dst_v, ones_v, deg_sp, sem):
    _deg_body(dst_hbm, zeros_hbm, ones_hbm, out_hbm, dst_v, ones_v, deg_sp, sem)


def _dinv_body(d0_ref, d1_ref, h_ref, dinv_ref, u_ref):
    deg = d0_ref[...] + d1_ref[...] + 1.0
    dinv = lax.rsqrt(deg)
    dinv_ref[...] = dinv
    u_ref[...] = dinv * h_ref[...]


def _dinv_u0(deg0, deg1, h):
    blk = 1000
    grid = N // blk
    return pl.pallas_call(
        _dinv_body,
        grid=(grid,),
        in_specs=[
            pl.BlockSpec((blk, 1), lambda i: (i, 0)),
            pl.BlockSpec((blk, 1), lambda i: (i, 0)),
            pl.BlockSpec((blk, 64), lambda i: (i, 0)),
        ],
        out_specs=[
            pl.BlockSpec((blk, 1), lambda i: (i, 0)),
            pl.BlockSpec((blk, 64), lambda i: (i, 0)),
        ],
        out_shape=[
            jax.ShapeDtypeStruct((N, 1), jnp.float32),
            jax.ShapeDtypeStruct((N, 64), jnp.float32),
        ],
    )(deg0, deg1, h)


def _prop_body(u_hbm, src_hbm, dst_hbm, zeros_hbm, out_hbm,
               src_v, dst_v, b0, b1, b2, b3, s_sp, m0, m1, m2, m3):
    bufs = (b0, b1, b2, b3)
    sems = (m0, m1, m2, m3)
    c = lax.axis_index("c")
    s = lax.axis_index("s")
    wid = c * 16 + s
    r0 = s * ROWS_PER_TILE
    pltpu.sync_copy(zeros_hbm.at[pl.ds(r0, ROWS_PER_TILE)],
                    s_sp.at[pl.ds(r0, ROWS_PER_TILE)])
    pltpu.sync_copy(src_hbm.at[wid], src_v)
    pltpu.sync_copy(dst_hbm.at[wid], dst_v)
    plsc.subcore_barrier()

    for b in range(NBUF):
        pltpu.async_copy(u_hbm.at[src_v.at[b]], bufs[b], sems[b])

    def blk(t, carry):
        base = t * NBUF
        for b in range(NBUF):
            j = base + b
            pltpu.make_async_copy(u_hbm.at[src_v.at[j]], bufs[b], sems[b]).wait()
            pltpu.sync_copy(bufs[b], s_sp.at[dst_v.at[j]], add=True)
            pltpu.async_copy(u_hbm.at[src_v.at[j + NBUF]], bufs[b], sems[b])
        return carry

    lax.fori_loop(0, NT - 1, blk, 0, unroll=False)

    base = (NT - 1) * NBUF
    for b in range(NBUF):
        j = base + b
        pltpu.make_async_copy(u_hbm.at[src_v.at[j]], bufs[b], sems[b]).wait()
        pltpu.sync_copy(bufs[b], s_sp.at[dst_v.at[j]], add=True)

    plsc.subcore_barrier()
    pltpu.sync_copy(s_sp.at[pl.ds(r0, ROWS_PER_TILE)],
                    out_hbm.at[c, pl.ds(r0, ROWS_PER_TILE)])


@functools.partial(
    pl.kernel,
    out_type=jax.ShapeDtypeStruct((2, N_PAD, N_OUT), jnp.float32),
    mesh=_mesh,
    compiler_params=pltpu.CompilerParams(use_tc_tiling_on_sc=False),
    scratch_types=[
        pltpu.VMEM((NB, B), jnp.int32),
        pltpu.VMEM((NB, B), jnp.int32),
        pltpu.VMEM((B, N_OUT), jnp.float32),
        pltpu.VMEM((B, N_OUT), jnp.float32),
        pltpu.VMEM((B, N_OUT), jnp.float32),
        pltpu.VMEM((B, N_OUT), jnp.float32),
        pltpu.VMEM_SHARED((N_PAD, N_OUT), jnp.float32),
        pltpu.SemaphoreType.DMA,
        pltpu.SemaphoreType.DMA,
        pltpu.SemaphoreType.DMA,
        pltpu.SemaphoreType.DMA,
    ],
)
def _prop_sc(u_hbm, src_hbm, dst_hbm, zeros_hbm, out_hbm,
             src_v, dst_v, b0, b1, b2, b3, s_sp, m0, m1, m2, m3):
    _prop_body(u_hbm, src_hbm, dst_hbm, zeros_hbm, out_hbm,
               src_v, dst_v, b0, b1, b2, b3, s_sp, m0, m1, m2, m3)


def _comb_body(s0_ref, s1_ref, u_ref, h_ref, dinv_ref, unew_ref, onew_ref):
    s = s0_ref[...] + s1_ref[...] + u_ref[...]
    onew = (1.0 - ALPHA) * dinv_ref[...] * s + ALPHA * h_ref[...]
    onew_ref[...] = onew
    unew_ref[...] = dinv_ref[...] * onew


def _combine(s0, s1, u, h, dinv):
    blk = 1000
    grid = N // blk
    return pl.pallas_call(
        _comb_body,
        grid=(grid,),
        in_specs=[
            pl.BlockSpec((blk, 64), lambda i: (i, 0)),
            pl.BlockSpec((blk, 64), lambda i: (i, 0)),
            pl.BlockSpec((blk, 64), lambda i: (i, 0)),
            pl.BlockSpec((blk, 64), lambda i: (i, 0)),
            pl.BlockSpec((blk, 1), lambda i: (i, 0)),
        ],
        out_specs=[
            pl.BlockSpec((blk, 64), lambda i: (i, 0)),
            pl.BlockSpec((blk, 64), lambda i: (i, 0)),
        ],
        out_shape=[
            jax.ShapeDtypeStruct((N, 64), jnp.float32),
            jax.ShapeDtypeStruct((N, 64), jnp.float32),
        ],
    )(s0, s1, u, h, dinv)


def kernel(x, edge_index, W1, b1, W2, b2):
    src = edge_index[0]
    dst = edge_index[1]
    pad = E_PAD - E
    src_p = jnp.concatenate([src, jnp.zeros((pad,), jnp.int32)])
    dst_p = jnp.concatenate([dst, jnp.full((pad,), TRASH, jnp.int32)])
    src_b = src_p.reshape(NW, NB, B)
    dst_b = dst_p.reshape(NW, NB, B)

    zeros64 = jnp.zeros((N_PAD, N_OUT), jnp.float32)
    zeros16 = jnp.zeros((N_PAD, 16), jnp.float32)
    ones16 = jnp.ones((B, 16), jnp.float32)

    h = _mlp(x, W1.T, b1.reshape(1, -1), W2.T, b2.reshape(1, -1))

    degp = _deg_sc(dst_b, zeros16, ones16)
    deg0 = degp[0, :N, :1]
    deg1 = degp[1, :N, :1]

    dinv, u = _dinv_u0(deg0, deg1, h)

    out = None
    for _ in range(K):
        sp = _prop_sc(u, src_b, dst_b, zeros64)
        u, out = _combine(sp[0, :N], sp[1, :N], u, h, dinv)
    return out


# speedup vs baseline: 3.6333x; 3.1739x over previous
import functools

import jax
import jax.numpy as jnp
from jax import lax
from jax.experimental import pallas as pl
from jax.experimental.pallas import tpu as pltpu
from jax.experimental.pallas import tpu_sc as plsc

N = 10000
N_PAD = 10112
TRASH = 10008
E = 320000
N_OUT = 64
K = 5
ALPHA = 0.1
NW = 32
B = 128
NB = 80
NBUF = 4
NT = NB // NBUF
E_PAD = NW * NB * B
ROWS_PER_TILE = N_PAD // 16

_mesh = plsc.VectorSubcoreMesh(core_axis_name="c", subcore_axis_name="s")


def _mlp_body(x_ref, w1_ref, b1_ref, w2_ref, b2_ref, o_ref):
    h = jnp.maximum(
        jnp.dot(x_ref[...], w1_ref[...], preferred_element_type=jnp.float32)
        + b1_ref[...],
        0.0,
    )
    o_ref[...] = (
        jnp.dot(h, w2_ref[...], preferred_element_type=jnp.float32) + b2_ref[...]
    )


def _mlp(x, w1t, b1, w2t, b2):
    blk = 1000
    grid = N // blk
    return pl.pallas_call(
        _mlp_body,
        grid=(grid,),
        in_specs=[
            pl.BlockSpec((blk, 128), lambda i: (i, 0)),
            pl.BlockSpec((128, 128), lambda i: (0, 0)),
            pl.BlockSpec((1, 128), lambda i: (0, 0)),
            pl.BlockSpec((128, 64), lambda i: (0, 0)),
            pl.BlockSpec((1, 64), lambda i: (0, 0)),
        ],
        out_specs=pl.BlockSpec((blk, 64), lambda i: (i, 0)),
        out_shape=jax.ShapeDtypeStruct((N, 64), jnp.float32),
    )(x, w1t, b1, w2t, b2)


def _deg_body(dst_hbm, zeros_hbm, ones_hbm, out_hbm, dst_v, ones_v, deg_sp, sem):
    c = lax.axis_index("c")
    s = lax.axis_index("s")
    wid = c * 16 + s
    r0 = s * ROWS_PER_TILE
    pltpu.sync_copy(zeros_hbm.at[pl.ds(r0, ROWS_PER_TILE)],
                    deg_sp.at[pl.ds(r0, ROWS_PER_TILE)])
    pltpu.sync_copy(dst_hbm.at[wid], dst_v)
    pltpu.sync_copy(ones_hbm, ones_v)
    plsc.subcore_barrier()

    def blk(j, carry):
        pltpu.sync_copy(ones_v, deg_sp.at[dst_v.at[j]], add=True)
        return carry

    lax.fori_loop(0, NB, blk, 0, unroll=False)
    plsc.subcore_barrier()
    pltpu.sync_copy(deg_sp.at[pl.ds(r0, ROWS_PER_TILE)],
                    out_hbm.at[c, pl.ds(r0, ROWS_PER_TILE)])


@functools.partial(
    pl.kernel,
    out_type=jax.ShapeDtypeStruct((2, N_PAD, 16), jnp.float32),
    mesh=_mesh,
    compiler_params=pltpu.CompilerParams(use_tc_tiling_on_sc=False),
    scratch_types=[
        pltpu.VMEM((NB, B), jnp.int32),
        pltpu.VMEM((B, 16), jnp.float32),
        pltpu.VMEM_SHARED((N_PAD, 16), jnp.float32),
        pltpu.SemaphoreType.DMA,
    ],
)
def _deg_sc(dst_hbm, zeros_hbm, ones_hbm, out_hbm, dst_v, ones_v, deg_sp, sem):
    _deg_body(dst_hbm, zeros_hbm, ones_hbm, out_hbm, dst_v, ones_v, deg_sp, sem)


def _dinv_body(d0_ref, d1_ref, h_ref, dinv_ref, u_ref):
    deg = d0_ref[...] + d1_ref[...] + 1.0
    dinv = lax.rsqrt(deg)
    dinv_ref[...] = dinv
    u_ref[...] = dinv * h_ref[...]


def _dinv_u0(deg0, deg1, h):
    blk = 1000
    grid = N // blk
    return pl.pallas_call(
        _dinv_body,
        grid=(grid,),
        in_specs=[
            pl.BlockSpec((blk, 1), lambda i: (i, 0)),
            pl.BlockSpec((blk, 1), lambda i: (i, 0)),
            pl.BlockSpec((blk, 64), lambda i: (i, 0)),
        ],
        out_specs=[
            pl.BlockSpec((blk, 1), lambda i: (i, 0)),
            pl.BlockSpec((blk, 64), lambda i: (i, 0)),
        ],
        out_shape=[
            jax.ShapeDtypeStruct((N, 1), jnp.float32),
            jax.ShapeDtypeStruct((N, 64), jnp.float32),
        ],
    )(deg0, deg1, h)


def _prop_body(u_hbm, src_hbm, dst_hbm, zeros_hbm, out_hbm,
               src_v, dst_v, b0, b1, b2, b3, s_sp, m0, m1, m2, m3):
    bufs = (b0, b1, b2, b3)
    sems = (m0, m1, m2, m3)
    c = lax.axis_index("c")
    s = lax.axis_index("s")
    wid = c * 16 + s
    r0 = s * ROWS_PER_TILE
    pltpu.sync_copy(zeros_hbm.at[pl.ds(r0, ROWS_PER_TILE)],
                    s_sp.at[pl.ds(r0, ROWS_PER_TILE)])
    pltpu.sync_copy(src_hbm.at[wid], src_v)
    pltpu.sync_copy(dst_hbm.at[wid], dst_v)
    plsc.subcore_barrier()

    for b in range(NBUF):
        pltpu.async_copy(u_hbm.at[src_v.at[b]], bufs[b], sems[b])

    def blk(t, carry):
        base = t * NBUF
        for b in range(NBUF):
            j = base + b
            pltpu.make_async_copy(u_hbm.at[src_v.at[j]], bufs[b], sems[b]).wait()
            pltpu.sync_copy(bufs[b], s_sp.at[dst_v.at[j]], add=True)
            pltpu.async_copy(u_hbm.at[src_v.at[j + NBUF]], bufs[b], sems[b])
        return carry

    lax.fori_loop(0, NT - 1, blk, 0, unroll=False)

    base = (NT - 1) * NBUF
    for b in range(NBUF):
        j = base + b
        pltpu.make_async_copy(u_hbm.at[src_v.at[j]], bufs[b], sems[b]).wait()
        pltpu.sync_copy(bufs[b], s_sp.at[dst_v.at[j]], add=True)

    plsc.subcore_barrier()
    pltpu.sync_copy(s_sp.at[pl.ds(r0, ROWS_PER_TILE)],
                    out_hbm.at[c, pl.ds(r0, ROWS_PER_TILE)])


@functools.partial(
    pl.kernel,
    out_type=jax.ShapeDtypeStruct((2, N_PAD, N_OUT), jnp.float32),
    mesh=_mesh,
    compiler_params=pltpu.CompilerParams(use_tc_tiling_on_sc=False),
    scratch_types=[
        pltpu.VMEM((NB, B), jnp.int32),
        pltpu.VMEM((NB, B), jnp.int32),
        pltpu.VMEM((B, N_OUT), jnp.float32),
        pltpu.VMEM((B, N_OUT), jnp.float32),
        pltpu.VMEM((B, N_OUT), jnp.float32),
        pltpu.VMEM((B, N_OUT), jnp.float32),
        pltpu.VMEM_SHARED((N_PAD, N_OUT), jnp.float32),
        pltpu.SemaphoreType.DMA,
        pltpu.SemaphoreType.DMA,
        pltpu.SemaphoreType.DMA,
        pltpu.SemaphoreType.DMA,
    ],
)
def _prop_sc(u_hbm, src_hbm, dst_hbm, zeros_hbm, out_hbm,
             src_v, dst_v, b0, b1, b2, b3, s_sp, m0, m1, m2, m3):
    _prop_body(u_hbm, src_hbm, dst_hbm, zeros_hbm, out_hbm,
               src_v, dst_v, b0, b1, b2, b3, s_sp, m0, m1, m2, m3)


def _comb_body(s0_ref, s1_ref, u_ref, h_ref, dinv_ref, unew_ref, onew_ref):
    s = s0_ref[...] + s1_ref[...] + u_ref[...]
    onew = (1.0 - ALPHA) * dinv_ref[...] * s + ALPHA * h_ref[...]
    onew_ref[...] = onew
    unew_ref[...] = dinv_ref[...] * onew


def _combine(s0, s1, u, h, dinv):
    blk = 1000
    grid = N // blk
    return pl.pallas_call(
        _comb_body,
        grid=(grid,),
        in_specs=[
            pl.BlockSpec((blk, 64), lambda i: (i, 0)),
            pl.BlockSpec((blk, 64), lambda i: (i, 0)),
            pl.BlockSpec((blk, 64), lambda i: (i, 0)),
            pl.BlockSpec((blk, 64), lambda i: (i, 0)),
            pl.BlockSpec((blk, 1), lambda i: (i, 0)),
        ],
        out_specs=[
            pl.BlockSpec((blk, 64), lambda i: (i, 0)),
            pl.BlockSpec((blk, 64), lambda i: (i, 0)),
        ],
        out_shape=[
            jax.ShapeDtypeStruct((N, 64), jnp.float32),
            jax.ShapeDtypeStruct((N, 64), jnp.float32),
        ],
    )(s0, s1, u, h, dinv)


def kernel(x, edge_index, W1, b1, W2, b2):
    src = edge_index[0]
    dst = edge_index[1]
    pad = E_PAD - E
    pad_idx = jnp.arange(pad, dtype=jnp.int32)
    src_p = jnp.concatenate([src, pad_idx % N])
    dst_p = jnp.concatenate([dst, N + pad_idx % (N_PAD - N)])
    src_b = src_p.reshape(NW, NB, B)
    dst_b = dst_p.reshape(NW, NB, B)

    zeros64 = jnp.zeros((N_PAD, N_OUT), jnp.float32)
    zeros16 = jnp.zeros((N_PAD, 16), jnp.float32)
    ones16 = jnp.ones((B, 16), jnp.float32)

    h = _mlp(x, W1.T, b1.reshape(1, -1), W2.T, b2.reshape(1, -1))

    degp = _deg_sc(dst_b, zeros16, ones16)
    deg0 = degp[0, :N, :1]
    deg1 = degp[1, :N, :1]

    dinv, u = _dinv_u0(deg0, deg1, h)

    out = None
    for _ in range(K):
        sp = _prop_sc(u, src_b, dst_b, zeros64)
        u, out = _combine(sp[0, :N], sp[1, :N], u, h, dinv)
    return out


# speedup vs baseline: 3.6493x; 1.0044x over previous
import functools

import jax
import jax.numpy as jnp
from jax import lax
from jax.experimental import pallas as pl
from jax.experimental.pallas import tpu as pltpu
from jax.experimental.pallas import tpu_sc as plsc

N = 10000
N_PAD = 10240
E = 320000
E2 = E + N
HALF = 32
K = 5
ALPHA = 0.1
B = 128
NBUF = 4
NB = 168
CB = 28
NCB = NB // CB
CBT = CB // NBUF
E_PAD = 16 * NB * B
RPT = N_PAD // 16

_mesh = plsc.VectorSubcoreMesh(core_axis_name="c", subcore_axis_name="s")


def _mlp_body(x_ref, w1_ref, b1_ref, w2_ref, b2_ref, o_ref):
    h = jnp.maximum(
        jnp.dot(x_ref[...], w1_ref[...], preferred_element_type=jnp.float32)
        + b1_ref[...],
        0.0,
    )
    o_ref[0] = (
        jnp.dot(h, w2_ref[0], preferred_element_type=jnp.float32) + b2_ref[0]
    )


def _mlp(x_pad, w1t, b1, w2t, b2):
    blk = 1024
    return pl.pallas_call(
        _mlp_body,
        grid=(2, N_PAD // blk),
        in_specs=[
            pl.BlockSpec((blk, 128), lambda c, i: (i, 0)),
            pl.BlockSpec((128, 128), lambda c, i: (0, 0)),
            pl.BlockSpec((1, 128), lambda c, i: (0, 0)),
            pl.BlockSpec((1, 128, HALF), lambda c, i: (c, 0, 0)),
            pl.BlockSpec((1, 1, HALF), lambda c, i: (c, 0, 0)),
        ],
        out_specs=pl.BlockSpec((1, blk, HALF), lambda c, i: (c, i, 0)),
        out_shape=jax.ShapeDtypeStruct((2, N_PAD, HALF), jnp.float32),
    )(x_pad, w1t, b1, w2t, b2)


def _rsqrt16(x):
    i = lax.bitcast_convert_type(x, jnp.int32)
    i = jnp.int32(0x5F3759DF) - (i >> 1)
    y = lax.bitcast_convert_type(i, jnp.float32)
    for _ in range(4):
        y = y * (1.5 - 0.5 * x * y * y)
    return y


def _appnp_body(h_hbm, src_hbm, dst_hbm, u_hbm, out_hbm,
                src_v, dst_v, b0, b1, b2, b3, ones_v, degr_v, dinv_v, c2_v,
                strip_v, s_sp, deg_sp, m0, m1, m2, m3):
    bufs = (b0, b1, b2, b3)
    sems = (m0, m1, m2, m3)
    c = lax.axis_index("c")
    s = lax.axis_index("s")
    r0 = s * RPT
    ubase = c * N_PAD + r0

    one16 = jnp.full((16,), 1.0, jnp.float32)
    zero16 = jnp.zeros((16,), jnp.float32)

    def fill_ones(j, carry):
        ones_v[j] = one16
        return carry

    lax.fori_loop(0, B, fill_ones, 0, unroll=False)

    def zero_deg(r, carry):
        degr_v[r] = zero16
        return carry

    lax.fori_loop(0, RPT, zero_deg, 0, unroll=False)

    def zero_strip(r, carry):
        strip_v[r, pl.ds(0, 16)] = zero16
        strip_v[r, pl.ds(16, 16)] = zero16
        return carry

    lax.fori_loop(0, RPT, zero_strip, 0, unroll=False)

    pltpu.sync_copy(degr_v, deg_sp.at[pl.ds(r0, RPT)])
    pltpu.sync_copy(strip_v, s_sp.at[pl.ds(r0, RPT)])
    plsc.subcore_barrier()

    def degchunk(q, carry):
        pltpu.sync_copy(dst_hbm.at[s, pl.ds(q * CB, CB)], dst_v)

        def degblk(j, cr):
            pltpu.sync_copy(ones_v, deg_sp.at[dst_v.at[j]], add=True)
            return cr

        lax.fori_loop(0, CB, degblk, 0, unroll=False)
        return carry

    lax.fori_loop(0, NCB, degchunk, 0, unroll=False)
    plsc.subcore_barrier()

    pltpu.sync_copy(deg_sp.at[pl.ds(r0, RPT)], degr_v)
    pltpu.sync_copy(h_hbm.at[pl.ds(ubase, RPT)], c2_v)

    def setup_row(r, carry):
        d = _rsqrt16(degr_v[r])
        dinv_v[r] = d
        h0 = c2_v[r, pl.ds(0, 16)]
        h1 = c2_v[r, pl.ds(16, 16)]
        c2_v[r, pl.ds(0, 16)] = ALPHA * h0
        c2_v[r, pl.ds(16, 16)] = ALPHA * h1
        strip_v[r, pl.ds(0, 16)] = d * h0
        strip_v[r, pl.ds(16, 16)] = d * h1
        return carry

    lax.fori_loop(0, RPT, setup_row, 0, unroll=False)
    pltpu.sync_copy(strip_v, u_hbm.at[pl.ds(ubase, RPT)])
    plsc.subcore_barrier()

    def hop(k, carry):
        def chunk(q, cq):
            pltpu.sync_copy(src_hbm.at[c, s, pl.ds(q * CB, CB)], src_v)
            pltpu.sync_copy(dst_hbm.at[s, pl.ds(q * CB, CB)], dst_v)
            for b in range(NBUF):
                pltpu.async_copy(u_hbm.at[src_v.at[b]], bufs[b], sems[b])

            def blk(t, cr):
                base = t * NBUF
                for b in range(NBUF):
                    j = base + b
                    pltpu.make_async_copy(u_hbm.at[src_v.at[j]], bufs[b],
                                          sems[b]).wait()
                    pltpu.sync_copy(bufs[b], s_sp.at[dst_v.at[j]], add=True)
                    pltpu.async_copy(u_hbm.at[src_v.at[j + NBUF]], bufs[b],
                                     sems[b])
                return cr

            lax.fori_loop(0, CBT - 1, blk, 0, unroll=False)
            base = (CBT - 1) * NBUF
            for b in range(NBUF):
                j = base + b
                pltpu.make_async_copy(u_hbm.at[src_v.at[j]], bufs[b],
                                      sems[b]).wait()
                pltpu.sync_copy(bufs[b], s_sp.at[dst_v.at[j]], add=True)
            return cq

        lax.fori_loop(0, NCB, chunk, 0, unroll=False)
        plsc.subcore_barrier()

        pltpu.sync_copy(s_sp.at[pl.ds(r0, RPT)], strip_v)
        last = k == K - 1

        def comb(r, cr):
            d = dinv_v[r]
            s0 = strip_v[r, pl.ds(0, 16)]
            s1 = strip_v[r, pl.ds(16, 16)]
            e0 = c2_v[r, pl.ds(0, 16)]
            e1 = c2_v[r, pl.ds(16, 16)]
            ds0 = (1.0 - ALPHA) * d * s0
            ds1 = (1.0 - ALPHA) * d * s1
            strip_v[r, pl.ds(0, 16)] = jnp.where(last, ds0 + e0,
                                                 d * (ds0 + e0))
            strip_v[r, pl.ds(16, 16)] = jnp.where(last, ds1 + e1,
                                                  d * (ds1 + e1))
            return cr

        lax.fori_loop(0, RPT, comb, 0, unroll=False)

        pltpu.sync_copy(strip_v, u_hbm.at[pl.ds(ubase, RPT)])

        @pl.when(last)
        def _():
            pltpu.sync_copy(strip_v, out_hbm.at[c, pl.ds(r0, RPT)])

        lax.fori_loop(0, RPT, zero_strip, 0, unroll=False)
        pltpu.sync_copy(strip_v, s_sp.at[pl.ds(r0, RPT)])
        plsc.subcore_barrier()
        return carry

    lax.fori_loop(0, K, hop, 0, unroll=False)


@functools.partial(
    pl.kernel,
    out_type=[
        jax.ShapeDtypeStruct((2 * N_PAD, HALF), jnp.float32),
        jax.ShapeDtypeStruct((2, N_PAD, HALF), jnp.float32),
    ],
    mesh=_mesh,
    compiler_params=pltpu.CompilerParams(use_tc_tiling_on_sc=False),
    scratch_types=[
        pltpu.VMEM((CB, B), jnp.int32),
        pltpu.VMEM((CB, B), jnp.int32),
        pltpu.VMEM((B, HALF), jnp.float32),
        pltpu.VMEM((B, HALF), jnp.float32),
        pltpu.VMEM((B, HALF), jnp.float32),
        pltpu.VMEM((B, HALF), jnp.float32),
        pltpu.VMEM((B, 16), jnp.float32),
        pltpu.VMEM((RPT, 16), jnp.float32),
        pltpu.VMEM((RPT, 16), jnp.float32),
        pltpu.VMEM((RPT, HALF), jnp.float32),
        pltpu.VMEM((RPT, HALF), jnp.float32),
        pltpu.VMEM_SHARED((N_PAD, HALF), jnp.float32),
        pltpu.VMEM_SHARED((N_PAD, 16), jnp.float32),
        pltpu.SemaphoreType.DMA,
        pltpu.SemaphoreType.DMA,
        pltpu.SemaphoreType.DMA,
        pltpu.SemaphoreType.DMA,
    ],
)
def _appnp_sc(h_hbm, src_hbm, dst_hbm, u_hbm, out_hbm,
              src_v, dst_v, b0, b1, b2, b3, ones_v, degr_v, dinv_v, c2_v,
              strip_v, s_sp, deg_sp, m0, m1, m2, m3):
    _appnp_body(h_hbm, src_hbm, dst_hbm, u_hbm, out_hbm,
                src_v, dst_v, b0, b1, b2, b3, ones_v, degr_v, dinv_v, c2_v,
                strip_v, s_sp, deg_sp, m0, m1, m2, m3)


def kernel(x, edge_index, W1, b1, W2, b2):
    loop = jnp.arange(N, dtype=jnp.int32)
    src = jnp.concatenate([edge_index[0], loop])
    dst = jnp.concatenate([edge_index[1], loop])
    pad = E_PAD - E2
    pad_idx = jnp.arange(pad, dtype=jnp.int32)
    src_p = jnp.concatenate([src, pad_idx % N]).reshape(16, NB, B)
    dst_p = jnp.concatenate([dst, N + pad_idx % (N_PAD - N)]).reshape(16, NB, B)
    src_b = jnp.stack([src_p, src_p + N_PAD])

    x_pad = jnp.concatenate([x, jnp.zeros((N_PAD - N, 128), jnp.float32)])

    w2s = W2.T.reshape(128, 2, HALF).transpose(1, 0, 2)
    b2s = b2.reshape(2, 1, HALF)
    h2 = _mlp(x_pad, W1.T, b1.reshape(1, -1), w2s, b2s)
    h_flat = h2.reshape(2 * N_PAD, HALF)

    _, outp = _appnp_sc(h_flat, src_b, dst_p)
    return jnp.concatenate([outp[0, :N], outp[1, :N]], axis=1)
